# Initial kernel scaffold; baseline (speedup 1.0000x reference)
import jax
import jax.numpy as jnp
from jax.experimental import pallas as pl


def kernel(x, Wt1, bt1, Wt2, bt2, Wt3, bt3, W1, b1, W2, b2, W3, b3, W4, b4, W5, b5, Wf1, bf1, Wf2, bf2, Wd1, bd1, Wd2, bd2, Wd3, bd3):
    raise NotImplementedError("write your pallas kernel here")


# speedup vs baseline: 9.8416x; 9.8416x over previous
import functools

import jax
import jax.numpy as jnp
from jax import lax
from jax.experimental import pallas as pl
from jax.experimental.pallas import tpu as pltpu
from jax.experimental.pallas import tpu_sc as plsc

KNN = 20
_HI = lax.Precision.DEFAULT


def _lrelu(v):
    return jnp.where(v >= 0, v, 0.2 * v)


def _knn_yc_body(xr_ref, xct_ref, w_ref, b_ref, idx_ref, y_ref, c_ref, *, C, R, N):
    b = pl.program_id(0)
    xr = xr_ref[0]
    xct = xct_ref[0]
    sq_r = jnp.sum(xr * xr, axis=1, keepdims=True)
    sq_c = jnp.sum(xct * xct, axis=0, keepdims=True)
    inner = lax.dot_general(xr, xct, (((1,), (0,)), ((), ())),
                            preferred_element_type=jnp.float32, precision=_HI)
    d = sq_r - 2.0 * inner + sq_c
    iota = lax.broadcasted_iota(jnp.int32, (R, N), 1)
    cols = []
    big_i = jnp.int32(2 ** 30)
    inf = jnp.float32(jnp.inf)
    for _ in range(KNN):
        m = jnp.min(d, axis=1, keepdims=True)
        cand = jnp.where(d == m, iota, big_i)
        amin = jnp.min(cand, axis=1, keepdims=True)
        d = jnp.where(iota == amin, inf, d)
        cols.append(amin)
    idx_ref[0] = jnp.concatenate(cols, axis=1) + b * N
    wt = w_ref[0:C, :]
    wb = w_ref[C:2 * C, :]
    y = lax.dot_general(xr, wt, (((1,), (0,)), ((), ())),
                        preferred_element_type=jnp.float32, precision=_HI)
    cb = lax.dot_general(xr, wb, (((1,), (0,)), ((), ())),
                         preferred_element_type=jnp.float32, precision=_HI)
    y_ref[0] = y
    c_ref[0] = cb - y + b_ref[:][None, :]


def _knn_yc(x, xt, W, bvec):
    B, N, C = x.shape
    C2, Cout = W.shape
    R = 256
    nb = N // R
    body = functools.partial(_knn_yc_body, C=C, R=R, N=N)
    return pl.pallas_call(
        body,
        grid=(B, nb),
        in_specs=[
            pl.BlockSpec((1, R, C), lambda b, r: (b, r, 0)),
            pl.BlockSpec((1, C, N), lambda b, r: (b, 0, 0)),
            pl.BlockSpec((C2, Cout), lambda b, r: (0, 0)),
            pl.BlockSpec((Cout,), lambda b, r: (0,)),
        ],
        out_specs=[
            pl.BlockSpec((1, R, KNN), lambda b, r: (b, r, 0)),
            pl.BlockSpec((1, R, Cout), lambda b, r: (b, r, 0)),
            pl.BlockSpec((1, R, Cout), lambda b, r: (b, r, 0)),
        ],
        out_shape=[
            jax.ShapeDtypeStruct((B, N, KNN), jnp.int32),
            jax.ShapeDtypeStruct((B, N, Cout), jnp.float32),
            jax.ShapeDtypeStruct((B, N, Cout), jnp.float32),
        ],
    )(x, xt, W, bvec)


def _gather_max(y_flat, gidx_flat, c_flat):
    TOT, Cout = y_flat.shape
    NW = 32
    per_w = TOT // NW
    P = 4
    iters = per_w // P
    mesh = plsc.VectorSubcoreMesh(core_axis_name="c", subcore_axis_name="s")

    @functools.partial(
        pl.kernel, mesh=mesh,
        compiler_params=pltpu.CompilerParams(use_tc_tiling_on_sc=False),
        out_type=jax.ShapeDtypeStruct((TOT, Cout), jnp.float32),
        scratch_types=[
            pltpu.VMEM((P * KNN,), jnp.int32),
            pltpu.VMEM((P * KNN, Cout), jnp.float32),
            pltpu.VMEM((P, Cout), jnp.float32),
            pltpu.VMEM((P, Cout), jnp.float32),
            pltpu.SemaphoreType.DMA,
        ],
    )
    def kfn(y_hbm, gidx_hbm, c_hbm, out_hbm, idx_v, rows_v, c_v, o_v, sem):
        wid = lax.axis_index("s") * 2 + lax.axis_index("c")
        base = wid * per_w

        def step(i, carry):
            p0 = base + i * P
            pltpu.sync_copy(gidx_hbm.at[pl.ds(p0 * KNN, P * KNN)], idx_v)
            pltpu.async_copy(y_hbm.at[idx_v], rows_v, sem).wait()
            pltpu.sync_copy(c_hbm.at[pl.ds(p0, P)], c_v)

            def chunk(co, c2):
                sl = pl.ds(co * 16, 16)
                for p in range(P):
                    acc = rows_v[p * KNN, sl]
                    for j in range(1, KNN):
                        acc = jnp.maximum(acc, rows_v[p * KNN + j, sl])
                    v = acc + c_v[p, sl]
                    o_v[p, sl] = jnp.where(v >= 0, v, 0.2 * v)
                return c2

            lax.fori_loop(0, Cout // 16, chunk, 0)
            pltpu.sync_copy(o_v, out_hbm.at[pl.ds(p0, P)])
            return carry

        lax.fori_loop(0, iters, step, 0)

    return kfn(y_flat, gidx_flat, c_flat)


def _tnet_gather(y_flat, gidx_flat, c_flat):
    TOT, Cw = y_flat.shape
    NW = 32
    per_w = TOT // NW
    P = 4
    iters = per_w // P
    mesh = plsc.VectorSubcoreMesh(core_axis_name="c", subcore_axis_name="s")

    @functools.partial(
        pl.kernel, mesh=mesh,
        compiler_params=pltpu.CompilerParams(use_tc_tiling_on_sc=False),
        out_type=jax.ShapeDtypeStruct((TOT * KNN, Cw), jnp.float32),
        scratch_types=[
            pltpu.VMEM((P * KNN,), jnp.int32),
            pltpu.VMEM((P * KNN, Cw), jnp.float32),
            pltpu.VMEM((P, Cw), jnp.float32),
            pltpu.SemaphoreType.DMA,
        ],
    )
    def kfn(y_hbm, gidx_hbm, c_hbm, out_hbm, idx_v, rows_v, c_v, sem):
        wid = lax.axis_index("s") * 2 + lax.axis_index("c")
        base = wid * per_w

        def step(i, carry):
            p0 = base + i * P
            pltpu.sync_copy(gidx_hbm.at[pl.ds(p0 * KNN, P * KNN)], idx_v)
            pltpu.async_copy(y_hbm.at[idx_v], rows_v, sem).wait()
            pltpu.sync_copy(c_hbm.at[pl.ds(p0, P)], c_v)

            def chunk(co, c2):
                sl = pl.ds(co * 16, 16)
                for p in range(P):
                    cv = c_v[p, sl]
                    for j in range(KNN):
                        v = rows_v[p * KNN + j, sl] + cv
                        rows_v[p * KNN + j, sl] = jnp.where(v >= 0, v, 0.2 * v)
                return c2

            lax.fori_loop(0, Cw // 16, chunk, 0)
            pltpu.sync_copy(rows_v, out_hbm.at[pl.ds(p0 * KNN, P * KNN)])
            return carry

        lax.fori_loop(0, iters, step, 0)

    return kfn(y_flat, gidx_flat, c_flat)


def _tnet_reduce_body(h_ref, w_ref, b_ref, g_ref):
    b = pl.program_id(0)
    j = pl.program_id(1)
    z = lax.dot_general(h_ref[...], w_ref[...], (((1,), (0,)), ((), ())),
                        preferred_element_type=jnp.float32, precision=_HI)
    z = _lrelu(z + b_ref[:][None, :])
    m = jnp.max(z, axis=0, keepdims=True)
    row = pl.ds(b, 1)

    @pl.when(j == 0)
    def _():
        g_ref[row, :] = m

    @pl.when(j > 0)
    def _():
        g_ref[row, :] = jnp.maximum(g_ref[row, :], m)


def _tnet_reduce(h, W, bvec, B):
    M, Cin = h.shape
    Cout = W.shape[1]
    RB = 4096
    nj = (M // B) // RB
    return pl.pallas_call(
        _tnet_reduce_body,
        grid=(B, nj),
        in_specs=[
            pl.BlockSpec((RB, Cin), lambda b, j, nj=nj: (b * nj + j, 0)),
            pl.BlockSpec((Cin, Cout), lambda b, j: (0, 0)),
            pl.BlockSpec((Cout,), lambda b, j: (0,)),
        ],
        out_specs=pl.BlockSpec((B, Cout), lambda b, j: (0, 0)),
        out_shape=jax.ShapeDtypeStruct((B, Cout), jnp.float32),
    )(h, W, bvec)


def _apply_t_body(x_ref, t_ref, o_ref):
    o_ref[0] = lax.dot_general(x_ref[0], t_ref[0], (((1,), (0,)), ((), ())),
                               preferred_element_type=jnp.float32, precision=_HI)


def _apply_t(x, T):
    B, N, C = x.shape
    return pl.pallas_call(
        _apply_t_body,
        grid=(B,),
        in_specs=[
            pl.BlockSpec((1, N, C), lambda b: (b, 0, 0)),
            pl.BlockSpec((1, C, C), lambda b: (b, 0, 0)),
        ],
        out_specs=pl.BlockSpec((1, N, C), lambda b: (b, 0, 0)),
        out_shape=jax.ShapeDtypeStruct((B, N, C), jnp.float32),
    )(x, T)


def _encode_body(x1_ref, x2_ref, x3_ref, x4_ref, w_ref, b_ref, o_ref):
    dg = (((1,), (0,)), ((), ()))
    h = lax.dot_general(x1_ref[0], w_ref[0:64, :], dg,
                        preferred_element_type=jnp.float32, precision=_HI)
    h += lax.dot_general(x2_ref[0], w_ref[64:128, :], dg,
                         preferred_element_type=jnp.float32, precision=_HI)
    h += lax.dot_general(x3_ref[0], w_ref[128:256, :], dg,
                         preferred_element_type=jnp.float32, precision=_HI)
    h += lax.dot_general(x4_ref[0], w_ref[256:512, :], dg,
                         preferred_element_type=jnp.float32, precision=_HI)
    h = _lrelu(h + b_ref[:][None, :])
    o_ref[pl.ds(pl.program_id(0), 1), :] = jnp.max(h, axis=0, keepdims=True)


def _encode(x1, x2, x3, x4, W5, b5):
    B, N, _ = x1.shape
    Cout = W5.shape[1]
    return pl.pallas_call(
        _encode_body,
        grid=(B,),
        in_specs=[
            pl.BlockSpec((1, N, 64), lambda b: (b, 0, 0)),
            pl.BlockSpec((1, N, 64), lambda b: (b, 0, 0)),
            pl.BlockSpec((1, N, 128), lambda b: (b, 0, 0)),
            pl.BlockSpec((1, N, 256), lambda b: (b, 0, 0)),
            pl.BlockSpec((512, Cout), lambda b: (0, 0)),
            pl.BlockSpec((Cout,), lambda b: (0,)),
        ],
        out_specs=pl.BlockSpec((B, Cout), lambda b: (0, 0)),
        out_shape=jax.ShapeDtypeStruct((B, Cout), jnp.float32),
    )(x1, x2, x3, x4, W5, b5)


def _decoder_body(code_ref, wf1, bf1, wf2, bf2, wd1, bd1, wd2, bd2, wd3, bd3,
                  o_ref):
    dg = (((1,), (0,)), ((), ()))

    def mm(a, b):
        return lax.dot_general(a, b, dg, preferred_element_type=jnp.float32,
                               precision=_HI)

    code = code_ref[...]
    h = jnp.maximum(mm(code, wf1[...]) + bf1[:][None, :], 0.0)
    code2 = mm(h, wf2[...]) + bf2[:][None, :]
    d1 = mm(code2, wd1[0:1024, :]) + mm(code2, wd1[1024:2048, :]) \
        + mm(code2, wd1[2048:3072, :]) + bd1[:][None, :]
    d1 = jnp.maximum(d1, 0.0)
    d2 = jnp.maximum(mm(d1, wd2[...]) + bd2[:][None, :], 0.0)
    o_ref[...] = mm(d2, wd3[...]) + bd3[:][None, :]


def _decoder(code, Wf1, bf1, Wf2, bf2, Wd1, bd1, Wd2, bd2, Wd3, bd3):
    B = code.shape[0]
    out_n = Wd3.shape[1]
    return pl.pallas_call(
        _decoder_body,
        out_shape=jax.ShapeDtypeStruct((B, out_n), jnp.float32),
    )(code, Wf1, bf1, Wf2, bf2, Wd1, bd1, Wd2, bd2, Wd3, bd3)


def kernel(x, Wt1, bt1, Wt2, bt2, Wt3, bt3, W1, b1, W2, b2, W3, b3, W4, b4,
           W5, b5, Wf1, bf1, Wf2, bf2, Wd1, bd1, Wd2, bd2, Wd3, bd3):
    B, N, _ = x.shape
    TOT = B * N

    def conv(xin, W, bvec):
        Cout = W.shape[1]
        xt = jnp.transpose(xin, (0, 2, 1))
        idx, y, c = _knn_yc(xin, xt, W, bvec)
        out = _gather_max(y.reshape(TOT, Cout), idx.reshape(TOT * KNN),
                          c.reshape(TOT, Cout))
        return out.reshape(B, N, Cout)

    xt0 = jnp.transpose(x, (0, 2, 1))
    idx0, y0, c0 = _knn_yc(x, xt0, Wt1, bt1)
    h = _tnet_gather(y0.reshape(TOT, 64), idx0.reshape(TOT * KNN),
                     c0.reshape(TOT, 64))
    g = _tnet_reduce(h, Wt2, bt2, B)
    T = (g @ Wt3 + bt3).reshape(B, 3, 3) + jnp.eye(3, dtype=x.dtype)
    tp = _apply_t(x, T)

    x1 = conv(tp, W1, b1)
    x2 = conv(x1, W2, b2)
    x3 = conv(x2, W3, b3)
    x4 = conv(x3, W4, b4)

    code = _encode(x1, x2, x3, x4, W5, b5)
    dec = _decoder(code, Wf1, bf1, Wf2, bf2, Wd1, bd1, Wd2, bd2, Wd3, bd3)
    decoded = jnp.transpose(dec.reshape(B, 3, N), (0, 2, 1))
    return decoded, tp


# speedup vs baseline: 13.0268x; 1.3236x over previous
import functools

import jax
import jax.numpy as jnp
from jax import lax
from jax.experimental import pallas as pl
from jax.experimental.pallas import tpu as pltpu
from jax.experimental.pallas import tpu_sc as plsc

KNN = 20
_HI = lax.Precision.DEFAULT


def _lrelu(v):
    return jnp.where(v >= 0, v, 0.2 * v)


def _knn_yc_body(xr_ref, xct_ref, w_ref, b_ref, idx_ref, y_ref, c_ref, *, C, R, N):
    b = pl.program_id(0)
    xr = xr_ref[0]
    xct = xct_ref[0]
    sq_r = jnp.sum(xr * xr, axis=1, keepdims=True)
    sq_c = jnp.sum(xct * xct, axis=0, keepdims=True)
    inner = lax.dot_general(xr, xct, (((1,), (0,)), ((), ())),
                            preferred_element_type=jnp.float32, precision=_HI)
    d = sq_r - 2.0 * inner + sq_c
    iota = lax.broadcasted_iota(jnp.int32, (R, N), 1)
    cols = []
    big_i = jnp.int32(2 ** 30)
    inf = jnp.float32(jnp.inf)
    for _ in range(KNN):
        m = jnp.min(d, axis=1, keepdims=True)
        cand = jnp.where(d == m, iota, big_i)
        amin = jnp.min(cand, axis=1, keepdims=True)
        d = jnp.where(iota == amin, inf, d)
        cols.append(amin)
    idx_ref[0] = jnp.concatenate(cols, axis=1) + b * N
    wt = w_ref[0:C, :]
    wb = w_ref[C:2 * C, :]
    y = lax.dot_general(xr, wt, (((1,), (0,)), ((), ())),
                        preferred_element_type=jnp.float32, precision=_HI)
    cb = lax.dot_general(xr, wb, (((1,), (0,)), ((), ())),
                         preferred_element_type=jnp.float32, precision=_HI)
    y_ref[0] = y
    c_ref[0] = cb - y + b_ref[:][None, :]


def _knn_yc(x, xt, W, bvec):
    B, N, C = x.shape
    C2, Cout = W.shape
    R = 256
    nb = N // R
    body = functools.partial(_knn_yc_body, C=C, R=R, N=N)
    return pl.pallas_call(
        body,
        grid=(B, nb),
        in_specs=[
            pl.BlockSpec((1, R, C), lambda b, r: (b, r, 0)),
            pl.BlockSpec((1, C, N), lambda b, r: (b, 0, 0)),
            pl.BlockSpec((C2, Cout), lambda b, r: (0, 0)),
            pl.BlockSpec((Cout,), lambda b, r: (0,)),
        ],
        out_specs=[
            pl.BlockSpec((1, R, KNN), lambda b, r: (b, r, 0)),
            pl.BlockSpec((1, R, Cout), lambda b, r: (b, r, 0)),
            pl.BlockSpec((1, R, Cout), lambda b, r: (b, r, 0)),
        ],
        out_shape=[
            jax.ShapeDtypeStruct((B, N, KNN), jnp.int32),
            jax.ShapeDtypeStruct((B, N, Cout), jnp.float32),
            jax.ShapeDtypeStruct((B, N, Cout), jnp.float32),
        ],
    )(x, xt, W, bvec)


def _gather_max(y_flat, gidx2d, c_flat):
    TOT, Cout = y_flat.shape
    NW = 32
    per_w = TOT // NW
    P = 4
    G = P * KNN
    iters = per_w // P
    mesh = plsc.VectorSubcoreMesh(core_axis_name="c", subcore_axis_name="s")
    sems = [pltpu.SemaphoreType.DMA] * 12

    @functools.partial(
        pl.kernel, mesh=mesh,
        compiler_params=pltpu.CompilerParams(use_tc_tiling_on_sc=False),
        out_type=jax.ShapeDtypeStruct((TOT, Cout), jnp.float32),
        scratch_types=[pltpu.VMEM((iters, G), jnp.int32)]
        + [pltpu.VMEM((G, Cout), jnp.float32)] * 4
        + [pltpu.VMEM((P, Cout), jnp.float32)] * 8
        + sems,
    )
    def kfn(y_hbm, gidx_hbm, c_hbm, out_hbm, idx_v,
            r0, r1, r2, r3, c0, c1, c2, c3, o0, o1, o2, o3, *sem):
        rows = [r0, r1, r2, r3]
        cbuf = [c0, c1, c2, c3]
        obuf = [o0, o1, o2, o3]
        sg = sem[0:4]
        sc_ = sem[4:8]
        so = sem[8:12]
        wid = lax.axis_index("s") * 2 + lax.axis_index("c")
        base = wid * per_w
        gbase = wid * iters

        def issue(t, v):
            pltpu.async_copy(y_hbm.at[idx_v.at[t]], rows[v], sg[v])
            pltpu.async_copy(c_hbm.at[pl.ds(base + t * P, P)], cbuf[v], sc_[v])

        pltpu.sync_copy(gidx_hbm.at[pl.ds(gbase, iters)], idx_v)
        issue(0, 0)
        issue(1, 1)

        def body(j, carry):
            for u in range(4):
                i = 4 * j + u

                @pl.when(j >= 1)
                def _():
                    pltpu.make_async_copy(obuf[u], out_hbm.at[pl.ds(0, P)],
                                          so[u]).wait()
                pltpu.make_async_copy(y_hbm.at[pl.ds(0, G)], rows[u],
                                      sg[u]).wait()
                pltpu.make_async_copy(c_hbm.at[pl.ds(0, P)], cbuf[u],
                                      sc_[u]).wait()

                def chunk(co, cc, u=u):
                    sl = pl.ds(co * 16, 16)
                    for p in range(P):
                        acc = rows[u][p * KNN, sl]
                        for k in range(1, KNN):
                            acc = jnp.maximum(acc, rows[u][p * KNN + k, sl])
                        v = acc + cbuf[u][p, sl]
                        obuf[u][p, sl] = jnp.where(v >= 0, v, 0.2 * v)
                    return cc

                lax.fori_loop(0, Cout // 16, chunk, 0)
                pltpu.async_copy(obuf[u], out_hbm.at[pl.ds(base + i * P, P)],
                                 so[u])
                t = i + 2
                v = (u + 2) % 4

                @pl.when(t < iters)
                def _(t=t, v=v):
                    issue(t, v)
            return carry

        lax.fori_loop(0, iters // 4, body, 0)
        for u in range(4):
            pltpu.make_async_copy(obuf[u], out_hbm.at[pl.ds(0, P)],
                                  so[u]).wait()

    return kfn(y_flat, gidx2d, c_flat)


def _tnet_gather(y_flat, gidx2d, c_flat):
    TOT, Cw = y_flat.shape
    NW = 32
    per_w = TOT // NW
    P = 4
    G = P * KNN
    iters = per_w // P
    mesh = plsc.VectorSubcoreMesh(core_axis_name="c", subcore_axis_name="s")

    @functools.partial(
        pl.kernel, mesh=mesh,
        compiler_params=pltpu.CompilerParams(use_tc_tiling_on_sc=False),
        out_type=jax.ShapeDtypeStruct((TOT * KNN, Cw), jnp.float32),
        scratch_types=[pltpu.VMEM((iters, G), jnp.int32)]
        + [pltpu.VMEM((G, Cw), jnp.float32)] * 4
        + [pltpu.VMEM((P, Cw), jnp.float32)] * 4
        + [pltpu.SemaphoreType.DMA] * 12,
    )
    def kfn(y_hbm, gidx_hbm, c_hbm, out_hbm, idx_v,
            r0, r1, r2, r3, c0, c1, c2, c3, *sem):
        rows = [r0, r1, r2, r3]
        cbuf = [c0, c1, c2, c3]
        sg = sem[0:4]
        sc_ = sem[4:8]
        so = sem[8:12]
        wid = lax.axis_index("s") * 2 + lax.axis_index("c")
        base = wid * per_w
        gbase = wid * iters

        def issue(t, v):
            pltpu.async_copy(y_hbm.at[idx_v.at[t]], rows[v], sg[v])
            pltpu.async_copy(c_hbm.at[pl.ds(base + t * P, P)], cbuf[v], sc_[v])

        pltpu.sync_copy(gidx_hbm.at[pl.ds(gbase, iters)], idx_v)
        issue(0, 0)
        issue(1, 1)

        def body(j, carry):
            for u in range(4):
                i = 4 * j + u
                pltpu.make_async_copy(y_hbm.at[pl.ds(0, G)], rows[u],
                                      sg[u]).wait()
                pltpu.make_async_copy(c_hbm.at[pl.ds(0, P)], cbuf[u],
                                      sc_[u]).wait()

                def chunk(co, cc, u=u):
                    sl = pl.ds(co * 16, 16)
                    for p in range(P):
                        cv = cbuf[u][p, sl]
                        for k in range(KNN):
                            v = rows[u][p * KNN + k, sl] + cv
                            rows[u][p * KNN + k, sl] = \
                                jnp.where(v >= 0, v, 0.2 * v)
                    return cc

                lax.fori_loop(0, Cw // 16, chunk, 0)
                pltpu.async_copy(rows[u],
                                 out_hbm.at[pl.ds((base + i * P) * KNN, G)],
                                 so[u])
                t = i + 2
                v = (u + 2) % 4

                @pl.when(t < iters)
                def _(t=t, v=v):
                    @pl.when(t >= 4)
                    def _():
                        pltpu.make_async_copy(
                            rows[v], out_hbm.at[pl.ds(0, G)], so[v]).wait()
                    issue(t, v)
            return carry

        lax.fori_loop(0, iters // 4, body, 0)
        for u in range(4):
            pltpu.make_async_copy(rows[u], out_hbm.at[pl.ds(0, G)],
                                  so[u]).wait()

    return kfn(y_flat, gidx2d, c_flat)


def _tnet_reduce_body(h_ref, w_ref, b_ref, g_ref):
    b = pl.program_id(0)
    j = pl.program_id(1)
    z = lax.dot_general(h_ref[...], w_ref[...], (((1,), (0,)), ((), ())),
                        preferred_element_type=jnp.float32, precision=_HI)
    z = _lrelu(z + b_ref[:][None, :])
    m = jnp.max(z, axis=0, keepdims=True)
    row = pl.ds(b, 1)

    @pl.when(j == 0)
    def _():
        g_ref[row, :] = m

    @pl.when(j > 0)
    def _():
        g_ref[row, :] = jnp.maximum(g_ref[row, :], m)


def _tnet_reduce(h, W, bvec, B):
    M, Cin = h.shape
    Cout = W.shape[1]
    RB = 4096
    nj = (M // B) // RB
    return pl.pallas_call(
        _tnet_reduce_body,
        grid=(B, nj),
        in_specs=[
            pl.BlockSpec((RB, Cin), lambda b, j, nj=nj: (b * nj + j, 0)),
            pl.BlockSpec((Cin, Cout), lambda b, j: (0, 0)),
            pl.BlockSpec((Cout,), lambda b, j: (0,)),
        ],
        out_specs=pl.BlockSpec((B, Cout), lambda b, j: (0, 0)),
        out_shape=jax.ShapeDtypeStruct((B, Cout), jnp.float32),
    )(h, W, bvec)


def _apply_t_body(x_ref, t_ref, o_ref):
    o_ref[0] = lax.dot_general(x_ref[0], t_ref[0], (((1,), (0,)), ((), ())),
                               preferred_element_type=jnp.float32, precision=_HI)


def _apply_t(x, T):
    B, N, C = x.shape
    return pl.pallas_call(
        _apply_t_body,
        grid=(B,),
        in_specs=[
            pl.BlockSpec((1, N, C), lambda b: (b, 0, 0)),
            pl.BlockSpec((1, C, C), lambda b: (b, 0, 0)),
        ],
        out_specs=pl.BlockSpec((1, N, C), lambda b: (b, 0, 0)),
        out_shape=jax.ShapeDtypeStruct((B, N, C), jnp.float32),
    )(x, T)


def _encode_body(x1_ref, x2_ref, x3_ref, x4_ref, w_ref, b_ref, o_ref):
    dg = (((1,), (0,)), ((), ()))
    h = lax.dot_general(x1_ref[0], w_ref[0:64, :], dg,
                        preferred_element_type=jnp.float32, precision=_HI)
    h += lax.dot_general(x2_ref[0], w_ref[64:128, :], dg,
                         preferred_element_type=jnp.float32, precision=_HI)
    h += lax.dot_general(x3_ref[0], w_ref[128:256, :], dg,
                         preferred_element_type=jnp.float32, precision=_HI)
    h += lax.dot_general(x4_ref[0], w_ref[256:512, :], dg,
                         preferred_element_type=jnp.float32, precision=_HI)
    h = _lrelu(h + b_ref[:][None, :])
    o_ref[pl.ds(pl.program_id(0), 1), :] = jnp.max(h, axis=0, keepdims=True)


def _encode(x1, x2, x3, x4, W5, b5):
    B, N, _ = x1.shape
    Cout = W5.shape[1]
    return pl.pallas_call(
        _encode_body,
        grid=(B,),
        in_specs=[
            pl.BlockSpec((1, N, 64), lambda b: (b, 0, 0)),
            pl.BlockSpec((1, N, 64), lambda b: (b, 0, 0)),
            pl.BlockSpec((1, N, 128), lambda b: (b, 0, 0)),
            pl.BlockSpec((1, N, 256), lambda b: (b, 0, 0)),
            pl.BlockSpec((512, Cout), lambda b: (0, 0)),
            pl.BlockSpec((Cout,), lambda b: (0,)),
        ],
        out_specs=pl.BlockSpec((B, Cout), lambda b: (0, 0)),
        out_shape=jax.ShapeDtypeStruct((B, Cout), jnp.float32),
    )(x1, x2, x3, x4, W5, b5)


def _decoder_body(code_ref, wf1, bf1, wf2, bf2, wd1, bd1, wd2, bd2, wd3, bd3,
                  o_ref):
    dg = (((1,), (0,)), ((), ()))

    def mm(a, b):
        return lax.dot_general(a, b, dg, preferred_element_type=jnp.float32,
                               precision=_HI)

    code = code_ref[...]
    h = jnp.maximum(mm(code, wf1[...]) + bf1[:][None, :], 0.0)
    code2 = mm(h, wf2[...]) + bf2[:][None, :]
    d1 = mm(code2, wd1[0:1024, :]) + mm(code2, wd1[1024:2048, :]) \
        + mm(code2, wd1[2048:3072, :]) + bd1[:][None, :]
    d1 = jnp.maximum(d1, 0.0)
    d2 = jnp.maximum(mm(d1, wd2[...]) + bd2[:][None, :], 0.0)
    o_ref[...] = mm(d2, wd3[...]) + bd3[:][None, :]


def _decoder(code, Wf1, bf1, Wf2, bf2, Wd1, bd1, Wd2, bd2, Wd3, bd3):
    B = code.shape[0]
    out_n = Wd3.shape[1]
    return pl.pallas_call(
        _decoder_body,
        out_shape=jax.ShapeDtypeStruct((B, out_n), jnp.float32),
    )(code, Wf1, bf1, Wf2, bf2, Wd1, bd1, Wd2, bd2, Wd3, bd3)


def kernel(x, Wt1, bt1, Wt2, bt2, Wt3, bt3, W1, b1, W2, b2, W3, b3, W4, b4,
           W5, b5, Wf1, bf1, Wf2, bf2, Wd1, bd1, Wd2, bd2, Wd3, bd3):
    B, N, _ = x.shape
    TOT = B * N

    def conv(xin, W, bvec):
        Cout = W.shape[1]
        xt = jnp.transpose(xin, (0, 2, 1))
        idx, y, c = _knn_yc(xin, xt, W, bvec)
        out = _gather_max(y.reshape(TOT, Cout), idx.reshape(-1, 4 * KNN),
                          c.reshape(TOT, Cout))
        return out.reshape(B, N, Cout)

    xt0 = jnp.transpose(x, (0, 2, 1))
    idx0, y0, c0 = _knn_yc(x, xt0, Wt1, bt1)
    h = _tnet_gather(y0.reshape(TOT, 64), idx0.reshape(-1, 4 * KNN),
                     c0.reshape(TOT, 64))
    g = _tnet_reduce(h, Wt2, bt2, B)
    T = (g @ Wt3 + bt3).reshape(B, 3, 3) + jnp.eye(3, dtype=x.dtype)
    tp = _apply_t(x, T)

    x1 = conv(tp, W1, b1)
    x2 = conv(x1, W2, b2)
    x3 = conv(x2, W3, b3)
    x4 = conv(x3, W4, b4)

    code = _encode(x1, x2, x3, x4, W5, b5)
    dec = _decoder(code, Wf1, bf1, Wf2, bf2, Wd1, bd1, Wd2, bd2, Wd3, bd3)
    decoded = jnp.transpose(dec.reshape(B, 3, N), (0, 2, 1))
    return decoded, tp


# speedup vs baseline: 19.0777x; 1.4645x over previous
import functools

import jax
import jax.numpy as jnp
from jax import lax
from jax.experimental import pallas as pl
from jax.experimental.pallas import tpu as pltpu
from jax.experimental.pallas import tpu_sc as plsc

KNN = 20
_HI = lax.Precision.DEFAULT


def _lrelu(v):
    return jnp.where(v >= 0, v, 0.2 * v)


def _knn_yc_body(xr_ref, xct_ref, w_ref, b_ref, idx_ref, y_ref, c_ref, *, C, R, N):
    b = pl.program_id(0)
    xr = xr_ref[0]
    xct = xct_ref[0]
    sq_r = jnp.sum(xr * xr, axis=1, keepdims=True)
    sq_c = jnp.sum(xct * xct, axis=0, keepdims=True)
    inner = lax.dot_general(xr, xct, (((1,), (0,)), ((), ())),
                            preferred_element_type=jnp.float32, precision=_HI)
    d = sq_r - 2.0 * inner + sq_c
    iota_f = lax.broadcasted_iota(jnp.int32, (R, N), 1).astype(jnp.float32)
    cols = []
    big_f = jnp.float32(1e9)
    inf = jnp.float32(jnp.inf)
    for _ in range(KNN):
        m = jnp.min(d, axis=1, keepdims=True)
        eq = d == m
        cand = jnp.where(eq, iota_f, big_f)
        amin = jnp.min(cand, axis=1, keepdims=True)
        d = jnp.where(eq, inf, d)
        cols.append(amin)
    idx = jnp.concatenate(cols, axis=1).astype(jnp.int32)
    idx_ref[0] = idx + b * N
    wt = w_ref[0:C, :]
    wb = w_ref[C:2 * C, :]
    y = lax.dot_general(xr, wt, (((1,), (0,)), ((), ())),
                        preferred_element_type=jnp.float32, precision=_HI)
    cb = lax.dot_general(xr, wb, (((1,), (0,)), ((), ())),
                         preferred_element_type=jnp.float32, precision=_HI)
    y_ref[0] = y
    c_ref[0] = cb - y + b_ref[:][None, :]


def _knn_yc(x, xt, W, bvec):
    B, N, C = x.shape
    C2, Cout = W.shape
    R = 256
    nb = N // R
    body = functools.partial(_knn_yc_body, C=C, R=R, N=N)
    return pl.pallas_call(
        body,
        grid=(B, nb),
        in_specs=[
            pl.BlockSpec((1, R, C), lambda b, r: (b, r, 0)),
            pl.BlockSpec((1, C, N), lambda b, r: (b, 0, 0)),
            pl.BlockSpec((C2, Cout), lambda b, r: (0, 0)),
            pl.BlockSpec((Cout,), lambda b, r: (0,)),
        ],
        out_specs=[
            pl.BlockSpec((1, R, KNN), lambda b, r: (b, r, 0)),
            pl.BlockSpec((1, R, Cout), lambda b, r: (b, r, 0)),
            pl.BlockSpec((1, R, Cout), lambda b, r: (b, r, 0)),
        ],
        out_shape=[
            jax.ShapeDtypeStruct((B, N, KNN), jnp.int32),
            jax.ShapeDtypeStruct((B, N, Cout), jnp.float32),
            jax.ShapeDtypeStruct((B, N, Cout), jnp.float32),
        ],
    )(x, xt, W, bvec)


def _gather_max(y_flat, gidx2d, c_flat):
    TOT, Cout = y_flat.shape
    NW = 32
    per_w = TOT // NW
    P = 4
    G = P * KNN
    iters = per_w // P
    mesh = plsc.VectorSubcoreMesh(core_axis_name="c", subcore_axis_name="s")
    sems = [pltpu.SemaphoreType.DMA] * 12

    @functools.partial(
        pl.kernel, mesh=mesh,
        compiler_params=pltpu.CompilerParams(use_tc_tiling_on_sc=False),
        out_type=jax.ShapeDtypeStruct((TOT, Cout), jnp.float32),
        scratch_types=[pltpu.VMEM((iters, G), jnp.int32)]
        + [pltpu.VMEM((G, Cout), jnp.float32)] * 4
        + [pltpu.VMEM((P, Cout), jnp.float32)] * 8
        + sems,
    )
    def kfn(y_hbm, gidx_hbm, c_hbm, out_hbm, idx_v,
            r0, r1, r2, r3, c0, c1, c2, c3, o0, o1, o2, o3, *sem):
        rows = [r0, r1, r2, r3]
        cbuf = [c0, c1, c2, c3]
        obuf = [o0, o1, o2, o3]
        sg = sem[0:4]
        sc_ = sem[4:8]
        so = sem[8:12]
        wid = lax.axis_index("s") * 2 + lax.axis_index("c")
        base = wid * per_w
        gbase = wid * iters

        def issue(t, v):
            pltpu.async_copy(y_hbm.at[idx_v.at[t]], rows[v], sg[v])
            pltpu.async_copy(c_hbm.at[pl.ds(base + t * P, P)], cbuf[v], sc_[v])

        pltpu.sync_copy(gidx_hbm.at[pl.ds(gbase, iters)], idx_v)
        issue(0, 0)
        issue(1, 1)

        def body(j, carry):
            for u in range(4):
                i = 4 * j + u

                @pl.when(j >= 1)
                def _():
                    pltpu.make_async_copy(obuf[u], out_hbm.at[pl.ds(0, P)],
                                          so[u]).wait()
                pltpu.make_async_copy(y_hbm.at[pl.ds(0, G)], rows[u],
                                      sg[u]).wait()
                pltpu.make_async_copy(c_hbm.at[pl.ds(0, P)], cbuf[u],
                                      sc_[u]).wait()

                def chunk(co, cc, u=u):
                    sl = pl.ds(co * 16, 16)
                    for p in range(P):
                        acc = rows[u][p * KNN, sl]
                        for k in range(1, KNN):
                            acc = jnp.maximum(acc, rows[u][p * KNN + k, sl])
                        v = acc + cbuf[u][p, sl]
                        obuf[u][p, sl] = jnp.where(v >= 0, v, 0.2 * v)
                    return cc

                lax.fori_loop(0, Cout // 16, chunk, 0)
                pltpu.async_copy(obuf[u], out_hbm.at[pl.ds(base + i * P, P)],
                                 so[u])
                t = i + 2
                v = (u + 2) % 4

                @pl.when(t < iters)
                def _(t=t, v=v):
                    issue(t, v)
            return carry

        lax.fori_loop(0, iters // 4, body, 0)
        for u in range(4):
            pltpu.make_async_copy(obuf[u], out_hbm.at[pl.ds(0, P)],
                                  so[u]).wait()

    return kfn(y_flat, gidx2d, c_flat)


def _tnet_gather(y_flat, gidx2d, c_flat):
    TOT, Cw = y_flat.shape
    NW = 32
    per_w = TOT // NW
    P = 4
    G = P * KNN
    iters = per_w // P
    mesh = plsc.VectorSubcoreMesh(core_axis_name="c", subcore_axis_name="s")

    @functools.partial(
        pl.kernel, mesh=mesh,
        compiler_params=pltpu.CompilerParams(use_tc_tiling_on_sc=False),
        out_type=jax.ShapeDtypeStruct((TOT * KNN, Cw), jnp.float32),
        scratch_types=[pltpu.VMEM((iters, G), jnp.int32)]
        + [pltpu.VMEM((G, Cw), jnp.float32)] * 4
        + [pltpu.VMEM((P, Cw), jnp.float32)] * 4
        + [pltpu.SemaphoreType.DMA] * 12,
    )
    def kfn(y_hbm, gidx_hbm, c_hbm, out_hbm, idx_v,
            r0, r1, r2, r3, c0, c1, c2, c3, *sem):
        rows = [r0, r1, r2, r3]
        cbuf = [c0, c1, c2, c3]
        sg = sem[0:4]
        sc_ = sem[4:8]
        so = sem[8:12]
        wid = lax.axis_index("s") * 2 + lax.axis_index("c")
        base = wid * per_w
        gbase = wid * iters

        def issue(t, v):
            pltpu.async_copy(y_hbm.at[idx_v.at[t]], rows[v], sg[v])
            pltpu.async_copy(c_hbm.at[pl.ds(base + t * P, P)], cbuf[v], sc_[v])

        pltpu.sync_copy(gidx_hbm.at[pl.ds(gbase, iters)], idx_v)
        issue(0, 0)
        issue(1, 1)

        def body(j, carry):
            for u in range(4):
                i = 4 * j + u
                pltpu.make_async_copy(y_hbm.at[pl.ds(0, G)], rows[u],
                                      sg[u]).wait()
                pltpu.make_async_copy(c_hbm.at[pl.ds(0, P)], cbuf[u],
                                      sc_[u]).wait()

                def chunk(co, cc, u=u):
                    sl = pl.ds(co * 16, 16)
                    for p in range(P):
                        cv = cbuf[u][p, sl]
                        for k in range(KNN):
                            v = rows[u][p * KNN + k, sl] + cv
                            rows[u][p * KNN + k, sl] = \
                                jnp.where(v >= 0, v, 0.2 * v)
                    return cc

                lax.fori_loop(0, Cw // 16, chunk, 0)
                pltpu.async_copy(rows[u],
                                 out_hbm.at[pl.ds((base + i * P) * KNN, G)],
                                 so[u])
                t = i + 2
                v = (u + 2) % 4

                @pl.when(t < iters)
                def _(t=t, v=v):
                    @pl.when(t >= 4)
                    def _():
                        pltpu.make_async_copy(
                            rows[v], out_hbm.at[pl.ds(0, G)], so[v]).wait()
                    issue(t, v)
            return carry

        lax.fori_loop(0, iters // 4, body, 0)
        for u in range(4):
            pltpu.make_async_copy(rows[u], out_hbm.at[pl.ds(0, G)],
                                  so[u]).wait()

    return kfn(y_flat, gidx2d, c_flat)


def _tnet_reduce_body(h_ref, w_ref, b_ref, g_ref):
    b = pl.program_id(0)
    j = pl.program_id(1)
    z = lax.dot_general(h_ref[...], w_ref[...], (((1,), (0,)), ((), ())),
                        preferred_element_type=jnp.float32, precision=_HI)
    z = _lrelu(z + b_ref[:][None, :])
    m = jnp.max(z, axis=0, keepdims=True)
    row = pl.ds(b, 1)

    @pl.when(j == 0)
    def _():
        g_ref[row, :] = m

    @pl.when(j > 0)
    def _():
        g_ref[row, :] = jnp.maximum(g_ref[row, :], m)


def _tnet_reduce(h, W, bvec, B):
    M, Cin = h.shape
    Cout = W.shape[1]
    RB = 4096
    nj = (M // B) // RB
    return pl.pallas_call(
        _tnet_reduce_body,
        grid=(B, nj),
        in_specs=[
            pl.BlockSpec((RB, Cin), lambda b, j, nj=nj: (b * nj + j, 0)),
            pl.BlockSpec((Cin, Cout), lambda b, j: (0, 0)),
            pl.BlockSpec((Cout,), lambda b, j: (0,)),
        ],
        out_specs=pl.BlockSpec((B, Cout), lambda b, j: (0, 0)),
        out_shape=jax.ShapeDtypeStruct((B, Cout), jnp.float32),
    )(h, W, bvec)


def _apply_t_body(x_ref, t_ref, o_ref):
    o_ref[0] = lax.dot_general(x_ref[0], t_ref[0], (((1,), (0,)), ((), ())),
                               preferred_element_type=jnp.float32, precision=_HI)


def _apply_t(x, T):
    B, N, C = x.shape
    return pl.pallas_call(
        _apply_t_body,
        grid=(B,),
        in_specs=[
            pl.BlockSpec((1, N, C), lambda b: (b, 0, 0)),
            pl.BlockSpec((1, C, C), lambda b: (b, 0, 0)),
        ],
        out_specs=pl.BlockSpec((1, N, C), lambda b: (b, 0, 0)),
        out_shape=jax.ShapeDtypeStruct((B, N, C), jnp.float32),
    )(x, T)


def _encode_body(x1_ref, x2_ref, x3_ref, x4_ref, w_ref, b_ref, o_ref):
    dg = (((1,), (0,)), ((), ()))
    h = lax.dot_general(x1_ref[0], w_ref[0:64, :], dg,
                        preferred_element_type=jnp.float32, precision=_HI)
    h += lax.dot_general(x2_ref[0], w_ref[64:128, :], dg,
                         preferred_element_type=jnp.float32, precision=_HI)
    h += lax.dot_general(x3_ref[0], w_ref[128:256, :], dg,
                         preferred_element_type=jnp.float32, precision=_HI)
    h += lax.dot_general(x4_ref[0], w_ref[256:512, :], dg,
                         preferred_element_type=jnp.float32, precision=_HI)
    h = _lrelu(h + b_ref[:][None, :])
    o_ref[pl.ds(pl.program_id(0), 1), :] = jnp.max(h, axis=0, keepdims=True)


def _encode(x1, x2, x3, x4, W5, b5):
    B, N, _ = x1.shape
    Cout = W5.shape[1]
    return pl.pallas_call(
        _encode_body,
        grid=(B,),
        in_specs=[
            pl.BlockSpec((1, N, 64), lambda b: (b, 0, 0)),
            pl.BlockSpec((1, N, 64), lambda b: (b, 0, 0)),
            pl.BlockSpec((1, N, 128), lambda b: (b, 0, 0)),
            pl.BlockSpec((1, N, 256), lambda b: (b, 0, 0)),
            pl.BlockSpec((512, Cout), lambda b: (0, 0)),
            pl.BlockSpec((Cout,), lambda b: (0,)),
        ],
        out_specs=pl.BlockSpec((B, Cout), lambda b: (0, 0)),
        out_shape=jax.ShapeDtypeStruct((B, Cout), jnp.float32),
    )(x1, x2, x3, x4, W5, b5)


def _decoder_body(code_ref, wf1, bf1, wf2, bf2, wd1, bd1, wd2, bd2, wd3, bd3,
                  o_ref):
    dg = (((1,), (0,)), ((), ()))

    def mm(a, b):
        return lax.dot_general(a, b, dg, preferred_element_type=jnp.float32,
                               precision=_HI)

    code = code_ref[...]
    h = jnp.maximum(mm(code, wf1[...]) + bf1[:][None, :], 0.0)
    code2 = mm(h, wf2[...]) + bf2[:][None, :]
    d1 = mm(code2, wd1[0:1024, :]) + mm(code2, wd1[1024:2048, :]) \
        + mm(code2, wd1[2048:3072, :]) + bd1[:][None, :]
    d1 = jnp.maximum(d1, 0.0)
    d2 = jnp.maximum(mm(d1, wd2[...]) + bd2[:][None, :], 0.0)
    o_ref[...] = mm(d2, wd3[...]) + bd3[:][None, :]


def _decoder(code, Wf1, bf1, Wf2, bf2, Wd1, bd1, Wd2, bd2, Wd3, bd3):
    B = code.shape[0]
    out_n = Wd3.shape[1]
    return pl.pallas_call(
        _decoder_body,
        out_shape=jax.ShapeDtypeStruct((B, out_n), jnp.float32),
    )(code, Wf1, bf1, Wf2, bf2, Wd1, bd1, Wd2, bd2, Wd3, bd3)


def kernel(x, Wt1, bt1, Wt2, bt2, Wt3, bt3, W1, b1, W2, b2, W3, b3, W4, b4,
           W5, b5, Wf1, bf1, Wf2, bf2, Wd1, bd1, Wd2, bd2, Wd3, bd3):
    B, N, _ = x.shape
    TOT = B * N

    def conv(xin, W, bvec):
        Cout = W.shape[1]
        xt = jnp.transpose(xin, (0, 2, 1))
        idx, y, c = _knn_yc(xin, xt, W, bvec)
        out = _gather_max(y.reshape(TOT, Cout), idx.reshape(-1, 4 * KNN),
                          c.reshape(TOT, Cout))
        return out.reshape(B, N, Cout)

    xt0 = jnp.transpose(x, (0, 2, 1))
    idx0, y0, c0 = _knn_yc(x, xt0, Wt1, bt1)
    h = _tnet_gather(y0.reshape(TOT, 64), idx0.reshape(-1, 4 * KNN),
                     c0.reshape(TOT, 64))
    g = _tnet_reduce(h, Wt2, bt2, B)
    T = (g @ Wt3 + bt3).reshape(B, 3, 3) + jnp.eye(3, dtype=x.dtype)
    tp = _apply_t(x, T)

    x1 = conv(tp, W1, b1)
    x2 = conv(x1, W2, b2)
    x3 = conv(x2, W3, b3)
    x4 = conv(x3, W4, b4)

    code = _encode(x1, x2, x3, x4, W5, b5)
    dec = _decoder(code, Wf1, bf1, Wf2, bf2, Wd1, bd1, Wd2, bd2, Wd3, bd3)
    decoded = jnp.transpose(dec.reshape(B, 3, N), (0, 2, 1))
    return decoded, tp
